# baseline (speedup 1.0000x reference)
import functools

import jax
import jax.numpy as jnp
from jax import lax
from jax.experimental import pallas as pl
from jax.experimental.pallas import tpu as pltpu
from jax.experimental.pallas import tpu_sc as plsc

NT = 16
NC = 2
CH = 128


def _sc_mesh():
    return plsc.VectorSubcoreMesh(core_axis_name="c", subcore_axis_name="s",
                                  num_cores=NC, num_subcores=NT)


def _make_hist(N, U, nch, ZC, OC, NR, interpret=False):
    @functools.partial(
        pl.kernel,
        out_type=jax.ShapeDtypeStruct((N, 16), jnp.float32),
        mesh=_sc_mesh(),
        scratch_types=[
            pltpu.VMEM((nch, CH), jnp.int32),
            pltpu.VMEM((CH, 16), jnp.float32),
            pltpu.VMEM_SHARED((NR, 16), jnp.float32),
            pltpu.SemaphoreType.DMA,
        ],
        interpret=interpret,
    )
    def hist(rows_hbm, ones_hbm, z16_hbm, deg_hbm, idx_v, ones_v, degbuf, sem):
        c = lax.axis_index("c")
        s = lax.axis_index("s")
        z0 = jnp.minimum(s * ZC, NR - ZC)
        pltpu.sync_copy(z16_hbm, degbuf.at[pl.ds(z0, ZC)])
        pltpu.sync_copy(rows_hbm.at[c, s], idx_v)
        pltpu.sync_copy(ones_hbm, ones_v)
        plsc.subcore_barrier()

        @pl.loop(0, nch)
        def _(j):
            pltpu.async_copy(ones_v, degbuf.at[idx_v.at[j]], sem, add=True)

        @pl.loop(0, nch)
        def _(j):
            pltpu.make_async_copy(ones_v, degbuf.at[idx_v.at[0]], sem).wait()

        plsc.subcore_barrier()
        l0 = jnp.minimum(s * OC, U - OC)
        pltpu.sync_copy(degbuf.at[pl.ds(l0, OC)],
                        deg_hbm.at[pl.ds(c * U + l0, OC)])

    return hist


def _make_spmm(N, U, D, nch, ZC, OC, NR, interpret=False):
    NB = 3

    @functools.partial(
        pl.kernel,
        out_type=jax.ShapeDtypeStruct((N, D), jnp.float32),
        mesh=_sc_mesh(),
        scratch_types=[
            pltpu.VMEM((nch, CH), jnp.int32),
            pltpu.VMEM((nch, CH), jnp.int32),
            pltpu.VMEM((CH, D), jnp.float32),
            pltpu.VMEM((CH, D), jnp.float32),
            pltpu.VMEM_SHARED((NR, D), jnp.float32),
            pltpu.SemaphoreType.DMA,
            pltpu.SemaphoreType.DMA,
            pltpu.SemaphoreType.DMA,
            pltpu.SemaphoreType.DMA,
        ],
        interpret=interpret,
    )
    def spmm(t_hbm, rows_hbm, cols_hbm, z_hbm, out_hbm,
             idx_r, idx_c, buf0, buf1, sbuf,
             g0, g1, s0, s1):
        bufs = (buf0, buf1)
        gsems = (g0, g1)
        ssems = (s0, s1)
        c = lax.axis_index("c")
        s = lax.axis_index("s")
        z0 = jnp.minimum(s * ZC, NR - ZC)
        pltpu.sync_copy(z_hbm, sbuf.at[pl.ds(z0, ZC)])
        pltpu.sync_copy(rows_hbm.at[c, s], idx_r)
        pltpu.sync_copy(cols_hbm.at[c, s], idx_c)
        pltpu.async_copy(t_hbm.at[idx_c.at[0]], bufs[0], gsems[0])
        plsc.subcore_barrier()

        @pl.loop(0, nch, step=2)
        def _(j0):
            for b in (0, 1):
                j = j0 + b
                pf = 1 - b

                @pl.when(j > 0)
                def _():
                    pltpu.make_async_copy(
                        bufs[pf], sbuf.at[idx_r.at[0]], ssems[pf]).wait()

                @pl.when(j + 1 < nch)
                def _():
                    pltpu.async_copy(t_hbm.at[idx_c.at[j + 1]],
                                     bufs[pf], gsems[pf])

                pltpu.make_async_copy(t_hbm.at[idx_c.at[0]], bufs[b],
                                      gsems[b]).wait()
                pltpu.async_copy(bufs[b], sbuf.at[idx_r.at[j]], ssems[b],
                                 add=True)

        pltpu.make_async_copy(bufs[1], sbuf.at[idx_r.at[0]],
                              ssems[1]).wait()
        plsc.subcore_barrier()
        l0 = jnp.minimum(s * OC, U - OC)
        pltpu.sync_copy(sbuf.at[pl.ds(l0, OC)],
                        out_hbm.at[pl.ds(c * U + l0, OC)])

    return spmm


def _tc_call(body, out_shapes, *args, interpret=False):
    return pl.pallas_call(
        body,
        out_shape=[jax.ShapeDtypeStruct(s, jnp.float32) for s in out_shapes],
        interpret=interpret,
    )(*args)


def _scale_body(U, deg_ref, u_ref, i_ref, dinv_ref, t0_ref):
    dinv = lax.rsqrt(deg_ref[...] + 1e-7)
    dinv_ref[...] = dinv
    t0_ref[:U, :] = u_ref[...] * dinv[:U, :1]
    t0_ref[U:, :] = i_ref[...] * dinv[U:, :1]


def _hgnn_body(U, u_ref, i_ref, eu_ref, ei_ref, h_ref):
    uE = u_ref[...]
    iE = i_ref[...]
    pu = lax.dot_general(uE, eu_ref[...], (((0,), (0,)), ((), ())),
                         preferred_element_type=jnp.float32)
    h_ref[:U, :] = jnp.dot(uE, pu, preferred_element_type=jnp.float32)
    pi = lax.dot_general(iE, ei_ref[...], (((0,), (0,)), ((), ())),
                         preferred_element_type=jnp.float32)
    h_ref[U:, :] = jnp.dot(iE, pi, preferred_element_type=jnp.float32)


def _mid_body(s1_ref, h1_ref, dinv_ref, tem1_ref, lats1_ref, t1_ref):
    dinv = dinv_ref[...][:, :1]
    tem1 = dinv * s1_ref[...]
    lats1 = tem1 + h1_ref[...]
    tem1_ref[...] = tem1
    lats1_ref[...] = lats1
    t1_ref[...] = dinv * lats1


def _final_body(U, s2_ref, h2_ref, tem1_ref, h1_ref, dinv_ref, u_ref, i_ref,
                tem2_ref, out_ref):
    dinv = dinv_ref[...][:, :1]
    tem2 = dinv * s2_ref[...]
    tem2_ref[...] = tem2
    acc = tem1_ref[...] + h1_ref[...] + tem2 + h2_ref[...]
    out_ref[:U, :] = u_ref[...] + acc[:U, :]
    out_ref[U:, :] = i_ref[...] + acc[U:, :]


def kernel(uEmbeds, iEmbeds, edge_index, interpret=False):
    U, D = uEmbeds.shape
    I = iEmbeds.shape[0]
    N = U + I
    E = edge_index.shape[0]
    assert U == I and D == 128

    nch = -(-E // (NT * CH))
    nch = 6 * (-(-nch // 6))
    Epad = NT * nch * CH
    NR = U + 8
    ZC = 8 * (-(-NR // (8 * NT)))
    OC = 8 * (-(-U // (8 * NT)))

    users = edge_index[:, 0]
    items = edge_index[:, 1]
    rows_p = jnp.pad(jnp.stack([users, items - U]), ((0, 0), (0, Epad - E)),
                     constant_values=U).reshape(NC, NT, nch, CH)
    cols_p = jnp.pad(jnp.stack([items, users]), ((0, 0), (0, Epad - E)),
                     constant_values=0).reshape(NC, NT, nch, CH)

    z16 = jnp.zeros((ZC, 16), jnp.float32)
    zD = jnp.zeros((ZC, D), jnp.float32)
    ones16 = jnp.ones((CH, 16), jnp.float32)

    hist = _make_hist(N, U, nch, ZC, OC, NR, interpret=interpret)
    spmm = _make_spmm(N, U, D, nch, ZC, OC, NR, interpret=interpret)

    deg = hist(rows_p, ones16, z16)
    h1, = _tc_call(functools.partial(_hgnn_body, U), [(N, D)],
                   uEmbeds, iEmbeds, uEmbeds, iEmbeds, interpret=interpret)
    dinv16, t0 = _tc_call(functools.partial(_scale_body, U), [(N, 16), (N, D)],
                          deg, uEmbeds, iEmbeds, interpret=interpret)

    s1 = spmm(t0, rows_p, cols_p, zD)
    tem1, lats1, t1 = _tc_call(_mid_body, [(N, D), (N, D), (N, D)],
                               s1, h1, dinv16, interpret=interpret)

    s2 = spmm(t1, rows_p, cols_p, zD)
    h2, = _tc_call(functools.partial(_hgnn_body, U), [(N, D)],
                   uEmbeds, iEmbeds, lats1[:U], lats1[U:], interpret=interpret)
    tem2, out = _tc_call(functools.partial(_final_body, U), [(N, D), (N, D)],
                         s2, h2, tem1, h1, dinv16, uEmbeds, iEmbeds,
                         interpret=interpret)

    embeds = jnp.concatenate([uEmbeds, iEmbeds], axis=0)
    gnnLats = jnp.stack([embeds, tem1, tem2])
    hyperLats = jnp.stack([embeds, h1, h2])
    return (out, gnnLats, hyperLats)


# speedup vs baseline: 3.7100x; 3.7100x over previous
import functools

import jax
import jax.numpy as jnp
from jax import lax
from jax.experimental import pallas as pl
from jax.experimental.pallas import tpu as pltpu
from jax.experimental.pallas import tpu_sc as plsc

NT = 16
NC = 2
CH = 128


def _sc_mesh():
    return plsc.VectorSubcoreMesh(core_axis_name="c", subcore_axis_name="s",
                                  num_cores=NC, num_subcores=NT)


def _make_hist(N, U, D, nch, ZC, OC, NR, interpret=False):
    @functools.partial(
        pl.kernel,
        out_type=jax.ShapeDtypeStruct((N, D), jnp.float32),
        mesh=_sc_mesh(),
        scratch_types=[
            pltpu.VMEM((nch, CH), jnp.int32),
            pltpu.VMEM((CH, D), jnp.float32),
            pltpu.VMEM_SHARED((NR, D), jnp.float32),
            pltpu.SemaphoreType.DMA,
            pltpu.SemaphoreType.DMA,
        ],
        interpret=interpret,
    )
    def hist(rows_hbm, ones_hbm, z16_hbm, deg_hbm, idx_v, ones_v, degbuf,
             sem0, sem1):
        sems = (sem0, sem1)
        c = lax.axis_index("c")
        s = lax.axis_index("s")
        z0 = jnp.minimum(s * ZC, NR - ZC)
        pltpu.sync_copy(z16_hbm, degbuf.at[pl.ds(z0, ZC)])
        pltpu.sync_copy(rows_hbm.at[c, s], idx_v)
        pltpu.sync_copy(ones_hbm, ones_v)
        plsc.subcore_barrier()

        @pl.loop(0, nch, step=2)
        def _(j0):
            for b in (0, 1):
                j = j0 + b

                @pl.when(j >= 2)
                def _():
                    pltpu.make_async_copy(ones_v, degbuf.at[idx_v.at[0]],
                                          sems[b]).wait()

                pltpu.async_copy(ones_v, degbuf.at[idx_v.at[j]], sems[b],
                                 add=True)

        for b in (0, 1):
            pltpu.make_async_copy(ones_v, degbuf.at[idx_v.at[0]],
                                  sems[b]).wait()
        plsc.subcore_barrier()
        l0 = jnp.minimum(s * OC, U - OC)
        pltpu.sync_copy(degbuf.at[pl.ds(l0, OC)],
                        deg_hbm.at[pl.ds(c * U + l0, OC)])

    return hist


def _make_spmm(N, U, D, nch, ZC, OC, NR, interpret=False):
    NB = 3

    @functools.partial(
        pl.kernel,
        out_type=jax.ShapeDtypeStruct((N, D), jnp.float32),
        mesh=_sc_mesh(),
        scratch_types=[
            pltpu.VMEM((nch, CH), jnp.int32),
            pltpu.VMEM((nch, CH), jnp.int32),
            pltpu.VMEM((CH, D), jnp.float32),
            pltpu.VMEM((CH, D), jnp.float32),
            pltpu.VMEM_SHARED((NR, D), jnp.float32),
            pltpu.SemaphoreType.DMA,
            pltpu.SemaphoreType.DMA,
            pltpu.SemaphoreType.DMA,
            pltpu.SemaphoreType.DMA,
        ],
        interpret=interpret,
    )
    def spmm(t_hbm, rows_hbm, cols_hbm, z_hbm, out_hbm,
             idx_r, idx_c, buf0, buf1, sbuf,
             g0, g1, s0, s1):
        bufs = (buf0, buf1)
        gsems = (g0, g1)
        ssems = (s0, s1)
        c = lax.axis_index("c")
        s = lax.axis_index("s")
        z0 = jnp.minimum(s * ZC, NR - ZC)
        pltpu.sync_copy(z_hbm, sbuf.at[pl.ds(z0, ZC)])
        pltpu.sync_copy(rows_hbm.at[c, s], idx_r)
        pltpu.sync_copy(cols_hbm.at[c, s], idx_c)
        pltpu.async_copy(t_hbm.at[idx_c.at[0]], bufs[0], gsems[0])
        plsc.subcore_barrier()

        @pl.loop(0, nch, step=2)
        def _(j0):
            for b in (0, 1):
                j = j0 + b
                pf = 1 - b

                @pl.when(j > 0)
                def _():
                    pltpu.make_async_copy(
                        bufs[pf], sbuf.at[idx_r.at[0]], ssems[pf]).wait()

                @pl.when(j + 1 < nch)
                def _():
                    pltpu.async_copy(t_hbm.at[idx_c.at[j + 1]],
                                     bufs[pf], gsems[pf])

                pltpu.make_async_copy(t_hbm.at[idx_c.at[0]], bufs[b],
                                      gsems[b]).wait()
                pltpu.async_copy(bufs[b], sbuf.at[idx_r.at[j]], ssems[b],
                                 add=True)

        pltpu.make_async_copy(bufs[1], sbuf.at[idx_r.at[0]],
                              ssems[1]).wait()
        plsc.subcore_barrier()
        l0 = jnp.minimum(s * OC, U - OC)
        pltpu.sync_copy(sbuf.at[pl.ds(l0, OC)],
                        out_hbm.at[pl.ds(c * U + l0, OC)])

    return spmm


def _tc_call(body, out_shapes, *args, interpret=False):
    return pl.pallas_call(
        body,
        out_shape=[jax.ShapeDtypeStruct(s, jnp.float32) for s in out_shapes],
        interpret=interpret,
    )(*args)


def _scale_body(U, deg_ref, u_ref, i_ref, dinv_ref, t0_ref):
    dinv = lax.rsqrt(deg_ref[...][:, :16] + 1e-7)
    dinv_ref[...] = dinv
    t0_ref[:U, :] = u_ref[...] * dinv[:U, :1]
    t0_ref[U:, :] = i_ref[...] * dinv[U:, :1]


def _hgnn_body(U, u_ref, i_ref, eu_ref, ei_ref, h_ref):
    uE = u_ref[...]
    iE = i_ref[...]
    pu = lax.dot_general(uE, eu_ref[...], (((0,), (0,)), ((), ())),
                         preferred_element_type=jnp.float32)
    h_ref[:U, :] = jnp.dot(uE, pu, preferred_element_type=jnp.float32)
    pi = lax.dot_general(iE, ei_ref[...], (((0,), (0,)), ((), ())),
                         preferred_element_type=jnp.float32)
    h_ref[U:, :] = jnp.dot(iE, pi, preferred_element_type=jnp.float32)


def _mid_body(s1_ref, h1_ref, dinv_ref, tem1_ref, lats1_ref, t1_ref):
    dinv = dinv_ref[...][:, :1]
    tem1 = dinv * s1_ref[...]
    lats1 = tem1 + h1_ref[...]
    tem1_ref[...] = tem1
    lats1_ref[...] = lats1
    t1_ref[...] = dinv * lats1


def _final_body(U, s2_ref, h2_ref, tem1_ref, h1_ref, dinv_ref, u_ref, i_ref,
                tem2_ref, out_ref):
    dinv = dinv_ref[...][:, :1]
    tem2 = dinv * s2_ref[...]
    tem2_ref[...] = tem2
    acc = tem1_ref[...] + h1_ref[...] + tem2 + h2_ref[...]
    out_ref[:U, :] = u_ref[...] + acc[:U, :]
    out_ref[U:, :] = i_ref[...] + acc[U:, :]


def kernel(uEmbeds, iEmbeds, edge_index, interpret=False):
    U, D = uEmbeds.shape
    I = iEmbeds.shape[0]
    N = U + I
    E = edge_index.shape[0]
    assert U == I and D == 128

    nch = 2 * (-(-E // (NT * CH * 2)))
    Epad = NT * nch * CH
    NR = U + 8
    ZC = 8 * (-(-NR // (8 * NT)))
    OC = 8 * (-(-U // (8 * NT)))

    users = edge_index[:, 0]
    items = edge_index[:, 1]
    pad_i = jnp.arange(Epad - E, dtype=jnp.int32)

    def _deal(idx, pad):
        full = jnp.concatenate([idx, pad])
        return full.reshape(nch, NT, CH).swapaxes(0, 1)

    rows_p = jnp.stack([_deal(users, U + pad_i % 8),
                        _deal(items - U, U + pad_i % 8)]).reshape(
                            NC, NT, nch, CH)
    cols_p = jnp.stack([_deal(items, pad_i % CH),
                        _deal(users, pad_i % CH)]).reshape(NC, NT, nch, CH)

    zD = jnp.zeros((ZC, D), jnp.float32)
    onesD = jnp.ones((CH, D), jnp.float32)

    hist = _make_hist(N, U, D, nch, ZC, OC, NR, interpret=interpret)
    spmm = _make_spmm(N, U, D, nch, ZC, OC, NR, interpret=interpret)

    deg = hist(rows_p, onesD, zD)
    h1, = _tc_call(functools.partial(_hgnn_body, U), [(N, D)],
                   uEmbeds, iEmbeds, uEmbeds, iEmbeds, interpret=interpret)
    dinv16, t0 = _tc_call(functools.partial(_scale_body, U), [(N, 16), (N, D)],
                          deg, uEmbeds, iEmbeds, interpret=interpret)

    s1 = spmm(t0, rows_p, cols_p, zD)
    tem1, lats1, t1 = _tc_call(_mid_body, [(N, D), (N, D), (N, D)],
                               s1, h1, dinv16, interpret=interpret)

    s2 = spmm(t1, rows_p, cols_p, zD)
    h2, = _tc_call(functools.partial(_hgnn_body, U), [(N, D)],
                   uEmbeds, iEmbeds, lats1[:U], lats1[U:], interpret=interpret)
    tem2, out = _tc_call(functools.partial(_final_body, U), [(N, D), (N, D)],
                         s2, h2, tem1, h1, dinv16, uEmbeds, iEmbeds,
                         interpret=interpret)

    embeds = jnp.concatenate([uEmbeds, iEmbeds], axis=0)
    gnnLats = jnp.stack([embeds, tem1, tem2])
    hyperLats = jnp.stack([embeds, h1, h2])
    return (out, gnnLats, hyperLats)
